# baseline (speedup 1.0000x reference)
import jax
import jax.numpy as jnp
from jax.experimental import pallas as pl
from jax.experimental.pallas import tpu as pltpu


def _body(x_ref, o_ref):
    o_ref[...] = x_ref[...] + 1.0


def kernel(x, time, embeddings):
    b, d, h, w = x.shape
    xr = x.reshape(32, 512, 4096)
    out = pl.pallas_call(
        _body,
        grid=(32,),
        in_specs=[pl.BlockSpec((1, 512, 4096), lambda i: (i, 0, 0))],
        out_specs=pl.BlockSpec((1, 512, 4096), lambda i: (i, 0, 0)),
        out_shape=jax.ShapeDtypeStruct((32, 512, 4096), x.dtype),
    )(xr)
    return out.reshape(b, d, h, w)


# speedup vs baseline: 2.5370x; 2.5370x over previous
import functools

import jax
import jax.numpy as jnp
from jax import lax
from jax.experimental import pallas as pl
from jax.experimental.pallas import tpu as pltpu
from jax.experimental.pallas import tpu_sc as plsc

_CH = 16


def kernel(x, time, embeddings):
    b, d, h, w = x.shape
    hw = h * w
    xr = x.reshape(b, d, hw)
    table = embeddings[:, :d]

    info = plsc.get_sparse_core_info()
    nc, ns = info.num_cores, info.num_subcores
    nw = nc * ns
    bpw = b // nw
    npb = d // _CH
    nch = bpw * npb
    nlv = hw // 16

    mesh = plsc.VectorSubcoreMesh(core_axis_name="c", subcore_axis_name="s")

    @functools.partial(
        pl.kernel,
        mesh=mesh,
        out_type=jax.ShapeDtypeStruct((b, d, hw), jnp.float32),
        compiler_params=pltpu.CompilerParams(needs_layout_passes=False),
        scratch_types=[
            pltpu.VMEM((nw, bpw), jnp.int32),
            pltpu.VMEM((bpw, d), jnp.float32),
            pltpu.VMEM((_CH, hw), jnp.float32),
            pltpu.VMEM((_CH, hw), jnp.float32),
            pltpu.VMEM((_CH, hw), jnp.float32),
            pltpu.VMEM((_CH, hw), jnp.float32),
            pltpu.SemaphoreType.DMA,
            pltpu.SemaphoreType.DMA,
            pltpu.SemaphoreType.DMA,
            pltpu.SemaphoreType.DMA,
            pltpu.SemaphoreType.DMA,
        ],
    )
    def sc_kernel(x_hbm, idx_hbm, tab_hbm, o_hbm, idx_v, erows,
                  xin0, xin1, xout0, xout1, gsem, isem0, isem1,
                  osem0, osem1):
        wid = lax.axis_index("s") * nc + lax.axis_index("c")
        base_b = wid * bpw

        pltpu.sync_copy(idx_hbm, idx_v)
        pltpu.async_copy(tab_hbm.at[idx_v.at[wid]], erows, gsem).wait()

        xins = (xin0, xin1)
        xouts = (xout0, xout1)
        isems = (isem0, isem1)
        osems = (osem0, osem1)

        def chunk_pos(i):
            bat_off = lax.div(i, npb)
            dd0 = lax.rem(i, npb) * _CH
            return bat_off, dd0

        def in_cp(i, slot):
            bat_off, dd0 = chunk_pos(i)
            return pltpu.make_async_copy(
                x_hbm.at[base_b + bat_off, pl.ds(dd0, _CH)],
                xins[slot], isems[slot],
            )

        def out_cp(i, slot):
            bat_off, dd0 = chunk_pos(i)
            return pltpu.make_async_copy(
                xouts[slot],
                o_hbm.at[base_b + bat_off, pl.ds(dd0, _CH)],
                osems[slot],
            )

        in_cp(0, 0).start()
        in_cp(1, 1).start()

        def do_chunk(i, slot):
            bat_off, dd0 = chunk_pos(i)
            xin, xout = xins[slot], xouts[slot]

            def row(r, _):
                aval = plsc.load_gather(
                    erows,
                    [
                        jnp.full((16,), bat_off, jnp.int32),
                        jnp.full((16,), dd0 + r, jnp.int32),
                    ],
                )
                for kk in range(nlv):
                    sl = pl.ds(kk * 16, 16)
                    xout[r, sl] = xin[r, sl] + aval
                return 0

            lax.fori_loop(0, _CH, row, 0, unroll=4)

        def pair(i2, _):
            for slot in range(2):
                i = i2 * 2 + slot
                in_cp(i, slot).wait()

                @pl.when(i >= 2)
                def _():
                    out_cp(i - 2, slot).wait()

                do_chunk(i, slot)
                out_cp(i, slot).start()

                @pl.when(i + 2 < nch)
                def _():
                    in_cp(i + 2, slot).start()

            return 0

        lax.fori_loop(0, nch // 2, pair, 0)
        out_cp(nch - 2, 0).wait()
        out_cp(nch - 1, 1).wait()

    t2 = time.astype(jnp.int32).reshape(nw, bpw)
    out = sc_kernel(xr, t2, table)
    return out.reshape(b, d, h, w)


# speedup vs baseline: 3.1862x; 1.2559x over previous
import functools

import jax
import jax.numpy as jnp
from jax import lax
from jax.experimental import pallas as pl
from jax.experimental.pallas import tpu as pltpu
from jax.experimental.pallas import tpu_sc as plsc

_CH = 16


def kernel(x, time, embeddings):
    b, d, h, w = x.shape
    hw = h * w
    xr = x.reshape(b, d, hw)
    table = embeddings[:, :d]

    info = plsc.get_sparse_core_info()
    nc, ns = info.num_cores, info.num_subcores
    nw = nc * ns
    bpw = b // nw
    npb = d // _CH
    nch = bpw * npb
    nlv = hw // 16

    mesh = plsc.VectorSubcoreMesh(core_axis_name="c", subcore_axis_name="s")

    @functools.partial(
        pl.kernel,
        mesh=mesh,
        out_type=jax.ShapeDtypeStruct((b, d, hw), jnp.float32),
        compiler_params=pltpu.CompilerParams(needs_layout_passes=False),
        scratch_types=[
            pltpu.VMEM((nw, bpw), jnp.int32),
            pltpu.VMEM((bpw, d), jnp.float32),
            pltpu.VMEM((_CH, hw), jnp.float32),
            pltpu.VMEM((_CH, hw), jnp.float32),
            pltpu.VMEM((_CH, hw), jnp.float32),
            pltpu.VMEM((_CH, hw), jnp.float32),
            pltpu.SemaphoreType.DMA,
            pltpu.SemaphoreType.DMA,
            pltpu.SemaphoreType.DMA,
            pltpu.SemaphoreType.DMA,
            pltpu.SemaphoreType.DMA,
            pltpu.SemaphoreType.DMA,
            pltpu.SemaphoreType.DMA,
            pltpu.SemaphoreType.DMA,
            pltpu.SemaphoreType.DMA,
        ],
    )
    def sc_kernel(x_hbm, idx_hbm, tab_hbm, o_hbm, idx_v, erows,
                  buf0, buf1, buf2, buf3, gsem, isem0, isem1, isem2,
                  isem3, osem0, osem1, osem2, osem3):
        wid = lax.axis_index("s") * nc + lax.axis_index("c")
        base_b = wid * bpw

        pltpu.sync_copy(idx_hbm, idx_v)
        pltpu.async_copy(tab_hbm.at[idx_v.at[wid]], erows, gsem).wait()

        bufs = (buf0, buf1, buf2, buf3)
        isems = (isem0, isem1, isem2, isem3)
        osems = (osem0, osem1, osem2, osem3)

        def chunk_pos(i):
            bat_off = lax.div(i, npb)
            dd0 = lax.rem(i, npb) * _CH
            return bat_off, dd0

        def in_cp(i, slot):
            bat_off, dd0 = chunk_pos(i)
            return pltpu.make_async_copy(
                x_hbm.at[base_b + bat_off, pl.ds(dd0, _CH)],
                bufs[slot], isems[slot],
            )

        def out_cp(i, slot):
            bat_off, dd0 = chunk_pos(i)
            return pltpu.make_async_copy(
                bufs[slot],
                o_hbm.at[base_b + bat_off, pl.ds(dd0, _CH)],
                osems[slot],
            )

        in_cp(0, 0).start()
        in_cp(1, 1).start()

        def do_chunk(i, slot):
            bat_off, dd0 = chunk_pos(i)
            buf = bufs[slot]

            def row(r, _):
                aval = plsc.load_gather(
                    erows,
                    [
                        jnp.full((16,), bat_off, jnp.int32),
                        jnp.full((16,), dd0 + r, jnp.int32),
                    ],
                )
                for kk in range(nlv):
                    sl = pl.ds(kk * 16, 16)
                    plsc.addupdate(buf.at[r, sl], aval)
                return 0

            lax.fori_loop(0, _CH, row, 0, unroll=4)

        def quad(i4, _):
            for slot in range(4):
                i = i4 * 4 + slot
                in_cp(i, slot).wait()
                do_chunk(i, slot)
                out_cp(i, slot).start()

                @pl.when(i >= 2)
                def _():
                    out_cp(i - 2, (slot - 2) % 4).wait()

                @pl.when(i + 2 < nch)
                def _():
                    in_cp(i + 2, (slot + 2) % 4).start()

            return 0

        lax.fori_loop(0, nch // 4, quad, 0)
        out_cp(nch - 2, (nch - 2) % 4).wait()
        out_cp(nch - 1, (nch - 1) % 4).wait()

    t2 = time.astype(jnp.int32).reshape(nw, bpw)
    out = sc_kernel(xr, t2, table)
    return out.reshape(b, d, h, w)
